# baseline (speedup 1.0000x reference)
import jax
import jax.numpy as jnp
from jax import lax
from jax.experimental import pallas as pl
from jax.experimental.pallas import tpu as pltpu
from jax.experimental.pallas import tpu_sc as plsc

B = 16384
F = 100
V = 1000000
NC = 2
NS = 16
NW = NC * NS
ROWS_PER_W = B // NW
RBLK = 128
NBLK = ROWS_PER_W // RBLK
EBLK = RBLK * F
NGATH = EBLK // 128


def _row_accumulate(gat_v, val_v, base):
    lane = jax.lax.iota(jnp.int32, 16)
    zero = jnp.zeros((16,), jnp.float32)
    accs = [zero, zero, zero, zero]
    for j in range(25):
        v = (gat_v[pl.ds(base + j * 16, 16)]
             * val_v[pl.ds(base + j * 16, 16)])
        e0 = j * 16
        r0 = e0 // F
        r1 = (e0 + 15) // F
        if r0 == r1:
            accs[r0] = accs[r0] + v
        else:
            cut = r1 * F - e0
            accs[r0] = accs[r0] + jnp.where(lane < cut, v, 0.0)
            accs[r1] = accs[r1] + jnp.where(lane < cut, 0.0, v)
    return accs


def _sc_body(idx_hbm, val_hbm, w_hbm, bias_hbm, out_hbm,
             idx_v, val_v, gat_v, out_v, bias_v, sem):
    wid = lax.axis_index("s") * NC + lax.axis_index("c")
    pltpu.sync_copy(bias_hbm, bias_v)

    def block(blk, carry):
        ebase = (wid * NBLK + blk) * EBLK
        pltpu.sync_copy(idx_hbm.at[pl.ds(ebase, EBLK)], idx_v)
        pltpu.sync_copy(val_hbm.at[pl.ds(ebase, EBLK)], val_v)

        def issue(j, c):
            pltpu.async_copy(w_hbm.at[idx_v.at[pl.ds(j * 128, 128)]],
                             gat_v.at[pl.ds(j * 128, 128)], sem)
            return c

        lax.fori_loop(0, NGATH, issue, 0)
        pltpu.make_async_copy(val_hbm.at[pl.ds(ebase, EBLK)], gat_v, sem).wait()

        lane = jax.lax.iota(jnp.int32, 16)
        bvec = bias_v[...]

        def sixteen_rows(rg, c):
            gbase = rg * (16 * F)
            outv = jnp.zeros((16,), jnp.float32)
            for u in range(4):
                accs = _row_accumulate(gat_v, val_v, gbase + u * 400)
                for k in range(4):
                    s = jnp.sum(accs[k])
                    outv = jnp.where(lane == (u * 4 + k), s, outv)
            out_v[pl.ds(rg * 16, 16)] = outv + bvec
            return c

        lax.fori_loop(0, RBLK // 16, sixteen_rows, 0)
        row0 = wid * ROWS_PER_W + blk * RBLK
        pltpu.sync_copy(out_v, out_hbm.at[pl.ds(row0, RBLK)])
        return carry

    lax.fori_loop(0, NBLK, block, 0)


@jax.jit
def _sc_call(idx_flat, val_flat, w0, bias16):
    mesh = plsc.VectorSubcoreMesh(core_axis_name="c", subcore_axis_name="s")
    f = pl.kernel(
        _sc_body,
        mesh=mesh,
        out_type=jax.ShapeDtypeStruct((B,), jnp.float32),
        scratch_types=[
            pltpu.VMEM((EBLK,), jnp.int32),
            pltpu.VMEM((EBLK,), jnp.float32),
            pltpu.VMEM((EBLK,), jnp.float32),
            pltpu.VMEM((RBLK,), jnp.float32),
            pltpu.VMEM((16,), jnp.float32),
            pltpu.SemaphoreType.DMA,
        ],
        compiler_params=pltpu.CompilerParams(needs_layout_passes=False),
    )
    return f(idx_flat, val_flat, w0, bias16)


def kernel(index_list, value_list, W, bias):
    idx_flat = index_list.reshape(B * F)
    val_flat = value_list.reshape(B * F)
    w0 = W.reshape(V)
    bias16 = jnp.broadcast_to(bias, (16,))
    res = _sc_call(idx_flat, val_flat, w0, bias16)
    return res.reshape(B, 1)


# speedup vs baseline: 1.0535x; 1.0535x over previous
import jax
import jax.numpy as jnp
from jax import lax
from jax.experimental import pallas as pl
from jax.experimental.pallas import tpu as pltpu
from jax.experimental.pallas import tpu_sc as plsc

B = 16384
F = 100
FP = 128
V = 1000000
NC = 2
NS = 16
NW = NC * NS
ROWS_PER_W = B // NW
RBLK = 128
NBLK = ROWS_PER_W // RBLK
GW = 104


def _sc_body(idx_hbm, val_hbm, w_hbm, bias_hbm, out_hbm,
             idx_v, val_v, gat_v, out_v, bias_v, drain_v, sem):
    wid = lax.axis_index("s") * NC + lax.axis_index("c")
    pltpu.sync_copy(bias_hbm, bias_v)
    lane = jax.lax.iota(jnp.int32, 16)
    tail_mask = lane >= 12

    def block(blk, carry):
        row0 = wid * ROWS_PER_W + blk * RBLK
        pltpu.sync_copy(idx_hbm.at[pl.ds(row0, RBLK)], idx_v)
        pltpu.sync_copy(val_hbm.at[pl.ds(row0, RBLK)], val_v)

        def issue(r, c):
            pltpu.async_copy(w_hbm.at[idx_v.at[r, pl.ds(0, F)]],
                             gat_v.at[r, pl.ds(0, F)], sem)
            return c

        lax.fori_loop(0, RBLK, issue, 0)
        pltpu.make_async_copy(w_hbm.at[pl.ds(0, RBLK * F)],
                              drain_v, sem).wait()

        bvec = bias_v[...]

        def sixteen_rows(rg, c):
            r0 = rg * 16
            outv = jnp.zeros((16,), jnp.float32)
            for k in range(16):
                r = r0 + k
                acc = (gat_v[r, pl.ds(0, 16)] * val_v[r, pl.ds(0, 16)]
                       + gat_v[r, pl.ds(16, 16)] * val_v[r, pl.ds(16, 16)])
                acc2 = (gat_v[r, pl.ds(32, 16)] * val_v[r, pl.ds(32, 16)]
                        + gat_v[r, pl.ds(48, 16)] * val_v[r, pl.ds(48, 16)])
                acc3 = (gat_v[r, pl.ds(64, 16)] * val_v[r, pl.ds(64, 16)]
                        + gat_v[r, pl.ds(80, 16)] * val_v[r, pl.ds(80, 16)])
                tail = jnp.where(tail_mask,
                                 gat_v[r, pl.ds(84, 16)]
                                 * val_v[r, pl.ds(84, 16)], 0.0)
                s = jnp.sum(acc + acc2 + acc3 + tail)
                outv = jnp.where(lane == k, s, outv)
            out_v[pl.ds(r0, 16)] = outv + bvec
            return c

        lax.fori_loop(0, RBLK // 16, sixteen_rows, 0)
        pltpu.sync_copy(out_v, out_hbm.at[pl.ds(row0, RBLK)])
        return carry

    lax.fori_loop(0, NBLK, block, 0)


@jax.jit
def _sc_call(idx_p, val_p, w0, bias16):
    mesh = plsc.VectorSubcoreMesh(core_axis_name="c", subcore_axis_name="s")
    f = pl.kernel(
        _sc_body,
        mesh=mesh,
        out_type=jax.ShapeDtypeStruct((B,), jnp.float32),
        scratch_types=[
            pltpu.VMEM((RBLK, FP), jnp.int32),
            pltpu.VMEM((RBLK, FP), jnp.float32),
            pltpu.VMEM((RBLK, GW), jnp.float32),
            pltpu.VMEM((RBLK,), jnp.float32),
            pltpu.VMEM((16,), jnp.float32),
            pltpu.VMEM((RBLK * F,), jnp.float32),
            pltpu.SemaphoreType.DMA,
        ],
        compiler_params=pltpu.CompilerParams(needs_layout_passes=False),
    )
    return f(idx_p, val_p, w0, bias16)


def kernel(index_list, value_list, W, bias):
    idx_p = jnp.pad(index_list, ((0, 0), (0, FP - F)))
    val_p = jnp.pad(value_list, ((0, 0), (0, FP - F)))
    w0 = W.reshape(V)
    bias16 = jnp.broadcast_to(bias, (16,))
    res = _sc_call(idx_p, val_p, w0, bias16)
    return res.reshape(B, 1)
